# baseline (speedup 1.0000x reference)
import functools

import jax
import jax.numpy as jnp
from jax import lax
from jax.experimental import pallas as pl
from jax.experimental.pallas import tpu as pltpu
from jax.experimental.pallas import tpu_sc as plsc

B_SZ, N_PTS, F_DIM = 8, 2048, 16
K_NN = 16
C_DIM = 32
ROWS = 256
N_HALF = 8
BH = B_SZ // N_HALF

NUM_WORKERS = 32
CHUNK_PTS = 128
IDX_PER_STREAM = 128


def _tc_body(x_ref, xta_ref, wp_ref, wq_ref, tvec_ref, idx_ref, p_ref, q_ref):
  x_rows = x_ref[0]
  xta = xta_ref[0]
  d = jnp.dot(x_rows, xta[:F_DIM, :], preferred_element_type=jnp.float32)
  d = d + xta[F_DIM:, :]
  colf = lax.broadcasted_iota(jnp.int32, (ROWS, N_PTS), 1).astype(jnp.float32)
  base = pl.program_id(0) * N_PTS
  inf = jnp.float32(3.0e38)
  row0 = pl.program_id(1) * ROWS
  selff = (lax.broadcasted_iota(jnp.int32, (ROWS, 1), 0)
           + row0).astype(jnp.float32)
  picks = [selff]
  d = jnp.where(colf == selff, inf, d)
  for t in range(K_NN - 1):
    m = jnp.min(d, axis=1, keepdims=True)
    eq = d == m
    sel = jnp.where(eq, colf, jnp.float32(2.0 * N_PTS))
    j = jnp.min(sel, axis=1, keepdims=True)
    picks.append(j)
    if t < K_NN - 2:
      d = jnp.where(eq, inf, d)
  idx_ref[...] = jnp.concatenate(picks, axis=1).astype(jnp.int32) + base
  p_ref[...] = (
      jnp.dot(x_rows, wp_ref[...], preferred_element_type=jnp.float32)
      + tvec_ref[...]
  )
  q_ref[...] = jnp.dot(x_rows, wq_ref[...], preferred_element_type=jnp.float32)


def _tc_knn(x, xtaug, wp, wq, tvec, boff):
  grid = (BH, N_PTS // ROWS)
  nblk = N_PTS // ROWS
  return pl.pallas_call(
      _tc_body,
      grid=grid,
      in_specs=[
          pl.BlockSpec((1, ROWS, F_DIM), lambda b, r: (b + boff, r, 0)),
          pl.BlockSpec((1, F_DIM + 1, N_PTS), lambda b, r: (b + boff, 0, 0)),
          pl.BlockSpec((F_DIM, C_DIM), lambda b, r: (0, 0)),
          pl.BlockSpec((F_DIM, C_DIM), lambda b, r: (0, 0)),
          pl.BlockSpec((1, C_DIM), lambda b, r: (0, 0)),
      ],
      out_specs=[
          pl.BlockSpec((ROWS, K_NN), lambda b, r: (b * nblk + r, 0)),
          pl.BlockSpec((ROWS, C_DIM), lambda b, r: (b * nblk + r, 0)),
          pl.BlockSpec((ROWS, C_DIM), lambda b, r: (b * nblk + r, 0)),
      ],
      out_shape=[
          jax.ShapeDtypeStruct((BH * N_PTS, K_NN), jnp.int32),
          jax.ShapeDtypeStruct((BH * N_PTS, C_DIM), jnp.float32),
          jax.ShapeDtypeStruct((BH * N_PTS, C_DIM), jnp.float32),
      ],
      compiler_params=pltpu.CompilerParams(
          dimension_semantics=("parallel", "parallel")),
  )(x, xtaug, wp, wq, tvec)


def _sc_body(idx_hbm, p_hbm, q_hbm, out_hbm, idx_v, p_v, out_v, rows_v, sem):
  npts = p_hbm.shape[0]
  pts_per_w = npts // NUM_WORKERS
  chunk_pts = min(CHUNK_PTS, pts_per_w)
  n_mega = pts_per_w // chunk_pts
  streams_per_mega = chunk_pts * K_NN // IDX_PER_STREAM
  idx_rows_per_w = pts_per_w * K_NN // IDX_PER_STREAM
  wid = lax.axis_index("s") * 2 + lax.axis_index("c")
  base = wid * pts_per_w
  pltpu.sync_copy(idx_hbm.at[pl.ds(wid * idx_rows_per_w, idx_rows_per_w)],
                  idx_v)
  pltpu.sync_copy(p_hbm.at[pl.ds(base, pts_per_w)], p_v)
  for m in range(n_mega):
    copies = []
    for j in range(streams_per_mega):
      copies.append(pltpu.async_copy(
          q_hbm.at[idx_v.at[m * streams_per_mega + j]],
          rows_v.at[pl.ds(j * IDX_PER_STREAM, IDX_PER_STREAM)],
          sem))
    for c in copies:
      c.wait()

    def pt_body(p, carry, m=m):
      pt = m * chunk_pts + p
      p0 = p_v[pt, pl.ds(0, 16)]
      p1 = p_v[pt, pl.ds(16, 16)]
      a0 = jnp.zeros((16,), jnp.float32)
      a1 = jnp.zeros((16,), jnp.float32)
      for k in range(K_NN):
        r0 = rows_v[p * K_NN + k, pl.ds(0, 16)]
        r1 = rows_v[p * K_NN + k, pl.ds(16, 16)]
        a0 = a0 + jnp.maximum(p0 + r0, 0.0)
        a1 = a1 + jnp.maximum(p1 + r1, 0.0)
      out_v[pt, pl.ds(0, 16)] = a0 * jnp.float32(1.0 / K_NN)
      out_v[pt, pl.ds(16, 16)] = a1 * jnp.float32(1.0 / K_NN)
      return carry

    lax.fori_loop(0, chunk_pts, pt_body, 0)
  pltpu.sync_copy(out_v, out_hbm.at[pl.ds(base, pts_per_w)])


def _sc_gather_mean(idx2d, p2d, q2d):
  npts = p2d.shape[0]
  pts_per_w = npts // NUM_WORKERS
  idx_rows_per_w = pts_per_w * K_NN // IDX_PER_STREAM
  mesh = plsc.VectorSubcoreMesh(core_axis_name="c", subcore_axis_name="s")
  f = functools.partial(
      pl.kernel,
      mesh=mesh,
      out_type=jax.ShapeDtypeStruct((npts, C_DIM), jnp.float32),
      scratch_types=[
          pltpu.VMEM((idx_rows_per_w, IDX_PER_STREAM), jnp.int32),
          pltpu.VMEM((pts_per_w, C_DIM), jnp.float32),
          pltpu.VMEM((pts_per_w, C_DIM), jnp.float32),
          pltpu.VMEM((min(CHUNK_PTS, pts_per_w) * K_NN, C_DIM), jnp.float32),
          pltpu.SemaphoreType.DMA,
      ],
      compiler_params=pltpu.CompilerParams(use_tc_tiling_on_sc=False),
  )(_sc_body)
  return f(idx2d, p2d, q2d)


def kernel(point_cloud, kernel0, bias0, gamma0, beta0, moving_mean0,
           moving_var0):
  x = point_cloud
  w = kernel0.reshape(2 * F_DIM, C_DIM)
  s = gamma0 * lax.rsqrt(moving_var0 + 1e-3)
  tvec = (bias0 - moving_mean0) * s + beta0
  wc, wd = w[:F_DIM], w[F_DIM:]
  wp = (wc - wd) * s[None, :]
  wq = wd * s[None, :]
  xt = jnp.transpose(x, (0, 2, 1))
  sq = jnp.sum(x * x, axis=2)
  xtaug = jnp.concatenate([xt * -2.0, sq[:, None, :]], axis=1)
  tv = tvec.reshape(1, C_DIM)
  outs = []
  for h in range(N_HALF):
    idx, p, q = _tc_knn(x, xtaug, wp, wq, tv, h * BH)
    outs.append(_sc_gather_mean(
        idx.reshape(BH * N_PTS * K_NN // IDX_PER_STREAM, IDX_PER_STREAM),
        p, q))
  return jnp.concatenate(outs, axis=0).reshape(B_SZ, N_PTS, C_DIM)


# speedup vs baseline: 1.0774x; 1.0774x over previous
import functools

import jax
import jax.numpy as jnp
from jax import lax
from jax.experimental import pallas as pl
from jax.experimental.pallas import tpu as pltpu
from jax.experimental.pallas import tpu_sc as plsc

B_SZ, N_PTS, F_DIM = 8, 2048, 16
K_NN = 16
C_DIM = 32
ROWS = 512
N_HALF = 4
BH = B_SZ // N_HALF

NUM_WORKERS = 32
CHUNK_PTS = 128
IDX_PER_STREAM = 128


def _tc_body(x_ref, xta_ref, wp_ref, wq_ref, tvec_ref, idx_ref, p_ref, q_ref):
  x_rows = x_ref[0]
  xta = xta_ref[0]
  d = jnp.dot(x_rows, xta[:F_DIM, :], preferred_element_type=jnp.float32)
  d = d + xta[F_DIM:, :]
  colf = lax.broadcasted_iota(jnp.int32, (ROWS, N_PTS), 1).astype(jnp.float32)
  base = pl.program_id(0) * N_PTS
  inf = jnp.float32(3.0e38)
  row0 = pl.program_id(1) * ROWS
  selff = (lax.broadcasted_iota(jnp.int32, (ROWS, 1), 0)
           + row0).astype(jnp.float32)
  picks = [selff]
  d = jnp.where(colf == selff, inf, d)
  for t in range(K_NN - 1):
    m = jnp.min(d, axis=1, keepdims=True)
    eq = d == m
    sel = jnp.where(eq, colf, jnp.float32(2.0 * N_PTS))
    j = jnp.min(sel, axis=1, keepdims=True)
    picks.append(j)
    if t < K_NN - 2:
      d = jnp.where(eq, inf, d)
  idx_ref[...] = jnp.concatenate(picks, axis=1).astype(jnp.int32) + base
  p_ref[...] = (
      jnp.dot(x_rows, wp_ref[...], preferred_element_type=jnp.float32)
      + tvec_ref[...]
  )
  q_ref[...] = jnp.dot(x_rows, wq_ref[...], preferred_element_type=jnp.float32)


def _tc_knn(x, xtaug, wp, wq, tvec, boff):
  grid = (BH, N_PTS // ROWS)
  nblk = N_PTS // ROWS
  return pl.pallas_call(
      _tc_body,
      grid=grid,
      in_specs=[
          pl.BlockSpec((1, ROWS, F_DIM), lambda b, r: (b + boff, r, 0)),
          pl.BlockSpec((1, F_DIM + 1, N_PTS), lambda b, r: (b + boff, 0, 0)),
          pl.BlockSpec((F_DIM, C_DIM), lambda b, r: (0, 0)),
          pl.BlockSpec((F_DIM, C_DIM), lambda b, r: (0, 0)),
          pl.BlockSpec((1, C_DIM), lambda b, r: (0, 0)),
      ],
      out_specs=[
          pl.BlockSpec((ROWS, K_NN), lambda b, r: (b * nblk + r, 0)),
          pl.BlockSpec((ROWS, C_DIM), lambda b, r: (b * nblk + r, 0)),
          pl.BlockSpec((ROWS, C_DIM), lambda b, r: (b * nblk + r, 0)),
      ],
      out_shape=[
          jax.ShapeDtypeStruct((BH * N_PTS, K_NN), jnp.int32),
          jax.ShapeDtypeStruct((BH * N_PTS, C_DIM), jnp.float32),
          jax.ShapeDtypeStruct((BH * N_PTS, C_DIM), jnp.float32),
      ],
      compiler_params=pltpu.CompilerParams(
          dimension_semantics=("parallel", "parallel")),
  )(x, xtaug, wp, wq, tvec)


def _sc_body(idx_hbm, p_hbm, q_hbm, out_hbm, idx_v, p_v, out_v, rows_v, sem):
  npts = p_hbm.shape[0]
  pts_per_w = npts // NUM_WORKERS
  chunk_pts = min(CHUNK_PTS, pts_per_w)
  n_mega = pts_per_w // chunk_pts
  streams_per_mega = chunk_pts * K_NN // IDX_PER_STREAM
  idx_rows_per_w = pts_per_w * K_NN // IDX_PER_STREAM
  wid = lax.axis_index("s") * 2 + lax.axis_index("c")
  base = wid * pts_per_w
  pltpu.sync_copy(idx_hbm.at[pl.ds(wid * idx_rows_per_w, idx_rows_per_w)],
                  idx_v)
  pltpu.sync_copy(p_hbm.at[pl.ds(base, pts_per_w)], p_v)
  for m in range(n_mega):
    copies = []
    for j in range(streams_per_mega):
      copies.append(pltpu.async_copy(
          q_hbm.at[idx_v.at[m * streams_per_mega + j]],
          rows_v.at[pl.ds(j * IDX_PER_STREAM, IDX_PER_STREAM)],
          sem))
    for c in copies:
      c.wait()

    def pt_body(p, carry, m=m):
      pt = m * chunk_pts + p
      p0 = p_v[pt, pl.ds(0, 16)]
      p1 = p_v[pt, pl.ds(16, 16)]
      a0 = jnp.zeros((16,), jnp.float32)
      a1 = jnp.zeros((16,), jnp.float32)
      for k in range(K_NN):
        r0 = rows_v[p * K_NN + k, pl.ds(0, 16)]
        r1 = rows_v[p * K_NN + k, pl.ds(16, 16)]
        a0 = a0 + jnp.maximum(p0 + r0, 0.0)
        a1 = a1 + jnp.maximum(p1 + r1, 0.0)
      out_v[pt, pl.ds(0, 16)] = a0 * jnp.float32(1.0 / K_NN)
      out_v[pt, pl.ds(16, 16)] = a1 * jnp.float32(1.0 / K_NN)
      return carry

    lax.fori_loop(0, chunk_pts, pt_body, 0)
  pltpu.sync_copy(out_v, out_hbm.at[pl.ds(base, pts_per_w)])


def _sc_gather_mean(idx2d, p2d, q2d):
  npts = p2d.shape[0]
  pts_per_w = npts // NUM_WORKERS
  idx_rows_per_w = pts_per_w * K_NN // IDX_PER_STREAM
  mesh = plsc.VectorSubcoreMesh(core_axis_name="c", subcore_axis_name="s")
  f = functools.partial(
      pl.kernel,
      mesh=mesh,
      out_type=jax.ShapeDtypeStruct((npts, C_DIM), jnp.float32),
      scratch_types=[
          pltpu.VMEM((idx_rows_per_w, IDX_PER_STREAM), jnp.int32),
          pltpu.VMEM((pts_per_w, C_DIM), jnp.float32),
          pltpu.VMEM((pts_per_w, C_DIM), jnp.float32),
          pltpu.VMEM((min(CHUNK_PTS, pts_per_w) * K_NN, C_DIM), jnp.float32),
          pltpu.SemaphoreType.DMA,
      ],
      compiler_params=pltpu.CompilerParams(use_tc_tiling_on_sc=False),
  )(_sc_body)
  return f(idx2d, p2d, q2d)


def kernel(point_cloud, kernel0, bias0, gamma0, beta0, moving_mean0,
           moving_var0):
  x = point_cloud
  w = kernel0.reshape(2 * F_DIM, C_DIM)
  s = gamma0 * lax.rsqrt(moving_var0 + 1e-3)
  tvec = (bias0 - moving_mean0) * s + beta0
  wc, wd = w[:F_DIM], w[F_DIM:]
  wp = (wc - wd) * s[None, :]
  wq = wd * s[None, :]
  xt = jnp.transpose(x, (0, 2, 1))
  sq = jnp.sum(x * x, axis=2)
  xtaug = jnp.concatenate([xt * -2.0, sq[:, None, :]], axis=1)
  tv = tvec.reshape(1, C_DIM)
  outs = []
  for h in range(N_HALF):
    idx, p, q = _tc_knn(x, xtaug, wp, wq, tv, h * BH)
    outs.append(_sc_gather_mean(
        idx.reshape(BH * N_PTS * K_NN // IDX_PER_STREAM, IDX_PER_STREAM),
        p, q))
  return jnp.concatenate(outs, axis=0).reshape(B_SZ, N_PTS, C_DIM)
